# baseline (speedup 1.0000x reference)
import jax
import jax.numpy as jnp
from jax import lax
from jax.experimental import pallas as pl
from jax.experimental.pallas import tpu as pltpu
from jax.experimental.pallas import tpu_sc as plsc

VOCAB = 100000
D = 128
B_TOTAL = 4096 * 200
SCALE = float(D) ** 0.5

NC, NS, L = 2, 16, 16
NW = NC * NS
ROWS_PER_W = B_TOTAL // NW
IROW = 128
CHUNK = 256
GPC = CHUNK // IROW
N_IDX_ROWS = ROWS_PER_W // IROW
N_CHUNKS = ROWS_PER_W // CHUNK


def _body(idx_hbm, table_hbm, out_hbm, idx_v, buf0, buf1, sem0, sem1):
    cid = lax.axis_index("c")
    sid = lax.axis_index("s")
    wid = sid * NC + cid

    pltpu.sync_copy(idx_hbm.at[pl.ds(wid * N_IDX_ROWS, N_IDX_ROWS)], idx_v)

    out_base = pl.multiple_of(wid * ROWS_PER_W, CHUNK)

    def start_gather(c, buf, sem):
        for j in range(GPC):
            pltpu.async_copy(table_hbm.at[idx_v.at[c * GPC + j]],
                             buf.at[pl.ds(j * IROW, IROW)], sem)

    def wait_gather(buf, sem):
        pltpu.make_async_copy(table_hbm.at[pl.ds(0, CHUNK)], buf, sem).wait()

    def scale_rows(buf):
        def row(r, _):
            for l in range(D // L):
                sl = pl.ds(l * L, L)
                buf[r, sl] = buf[r, sl] * SCALE
            return 0
        lax.fori_loop(0, CHUNK, row, 0, unroll=4)

    def flush(c, buf):
        scale_rows(buf)
        pltpu.sync_copy(buf, out_hbm.at[pl.ds(out_base + c * CHUNK, CHUNK)])

    start_gather(0, buf0, sem0)

    def step(g, _):
        c0 = g * 2
        start_gather(c0 + 1, buf1, sem1)
        wait_gather(buf0, sem0)
        flush(c0, buf0)

        @pl.when(c0 + 2 < N_CHUNKS)
        def _():
            start_gather(c0 + 2, buf0, sem0)

        wait_gather(buf1, sem1)
        flush(c0 + 1, buf1)
        return 0

    lax.fori_loop(0, N_CHUNKS // 2, step, 0)


@jax.jit
def _embed(idx2d, embeddings):
    mesh = plsc.VectorSubcoreMesh(core_axis_name="c", subcore_axis_name="s")
    run = pl.kernel(
        _body,
        out_type=jax.ShapeDtypeStruct((B_TOTAL, D), jnp.float32),
        mesh=mesh,
        scratch_types=[
            pltpu.VMEM((N_IDX_ROWS, IROW), jnp.int32),
            pltpu.VMEM((CHUNK, D), jnp.float32),
            pltpu.VMEM((CHUNK, D), jnp.float32),
            pltpu.SemaphoreType.DMA,
            pltpu.SemaphoreType.DMA,
        ],
    )
    return run(idx2d, embeddings)


def kernel(inputs, embeddings):
    idx2d = inputs.astype(jnp.int32).reshape(B_TOTAL // IROW, IROW)
    out = _embed(idx2d, embeddings)
    return out.reshape(inputs.shape[0], inputs.shape[1], D)
